# baseline (speedup 1.0000x reference)
import dataclasses
import functools

import jax
import jax.numpy as jnp
from jax import lax
from jax.experimental import pallas as pl
from jax.experimental.pallas import tpu as pltpu
from jax.experimental.pallas import tpu_sc as plsc

B = 16384
D = 32
F = 32
NC, NS, L = 2, 16, 16
NW = NC * NS
BPW = B // NW
CHUNK = 128
NCHUNK = BPW // CHUNK


def _make_sc_kernel():
    mesh = plsc.VectorSubcoreMesh(core_axis_name="c", subcore_axis_name="s")
    cp = pltpu.CompilerParams(needs_layout_passes=False,
                              use_tc_tiling_on_sc=False)

    @functools.partial(
        pl.kernel,
        mesh=mesh,
        out_type=jax.ShapeDtypeStruct((B,), jnp.float32),
        scratch_types=[
            pltpu.VMEM((NCHUNK, CHUNK), jnp.int32),
            pltpu.VMEM((NCHUNK, CHUNK), jnp.int32),
            pltpu.VMEM((BPW, D), jnp.float32),
            pltpu.VMEM((BPW, D), jnp.float32),
            pltpu.VMEM((BPW, F), jnp.float32),
            pltpu.VMEM((80,), jnp.float32),
            pltpu.VMEM((BPW,), jnp.float32),
            pltpu.SemaphoreType.DMA,
        ],
        compiler_params=cp,
    )
    def k(ui_hbm, mi_hbm, mf_hbm, ut_hbm, mt_hbm, wb_hbm, o_hbm,
          ui_v, mi_v, ue_v, me_v, mf_v, wb_v, o_v, sem):
        wid = lax.axis_index("s") * NC + lax.axis_index("c")
        base = wid * BPW

        pltpu.sync_copy(ui_hbm.at[pl.ds(wid * NCHUNK, NCHUNK)], ui_v)
        pltpu.sync_copy(mi_hbm.at[pl.ds(wid * NCHUNK, NCHUNK)], mi_v)

        copies = []
        for j in range(NCHUNK):
            copies.append(pltpu.async_copy(
                ut_hbm.at[ui_v.at[j]], ue_v.at[pl.ds(j * CHUNK, CHUNK)], sem))
            copies.append(pltpu.async_copy(
                mt_hbm.at[mi_v.at[j]], me_v.at[pl.ds(j * CHUNK, CHUNK)], sem))
        pltpu.sync_copy(mf_hbm.at[pl.ds(base, BPW)], mf_v)
        pltpu.sync_copy(wb_hbm, wb_v)
        for c in copies:
            c.wait()

        w1a = wb_v[pl.ds(0, L)]
        w1b = wb_v[pl.ds(L, L)]
        w2a = wb_v[pl.ds(2 * L, L)]
        w2b = wb_v[pl.ds(3 * L, L)]
        bias = wb_v[pl.ds(4 * L, L)][0]
        lanes = lax.iota(jnp.int32, L)

        @pl.loop(0, BPW // L)
        def _(g):
            r0 = g * L
            res = jnp.zeros((L,), jnp.float32)
            for k in range(L):
                i = r0 + k
                v = (ue_v[i, pl.ds(0, L)] * me_v[i, pl.ds(0, L)] * w1a
                     + ue_v[i, pl.ds(L, L)] * me_v[i, pl.ds(L, L)] * w1b
                     + mf_v[i, pl.ds(0, L)] * w2a
                     + mf_v[i, pl.ds(L, L)] * w2b)
                res = jnp.where(lanes == k, jnp.sum(v), res)
            o_v[pl.ds(r0, L)] = res + bias

        pltpu.sync_copy(o_v, o_hbm.at[pl.ds(base, BPW)])

    return k


_sc_forward = _make_sc_kernel()


def kernel(user_idx, movie_idx, movie_feats, user_table, movie_table, W, b):
    ui = user_idx.astype(jnp.int32).reshape(B // CHUNK, CHUNK)
    mi = movie_idx.astype(jnp.int32).reshape(B // CHUNK, CHUNK)
    wb = jnp.zeros((80,), jnp.float32).at[:64].set(W[:, 0]).at[64].set(b[0])
    return _sc_forward(ui, mi, movie_feats, user_table, movie_table, wb)
